# baseline (speedup 1.0000x reference)
import functools

import jax
import jax.numpy as jnp
from jax import lax
from jax.experimental import pallas as pl
from jax.experimental.pallas import tpu as pltpu
from jax.experimental.pallas import tpu_sc as plsc

BATCH = 16384
FIELDS = 26
D = 64
W = 128
V = 1000000
B = BATCH * FIELDS
NW = 32
BPW = B // NW
CH = 128
NCH = BPW // CH
NTB = V // CH
VREM = V - NTB * CH


def _build_tpose():
    mesh = plsc.VectorSubcoreMesh(core_axis_name="c", subcore_axis_name="s")

    @functools.partial(
        pl.kernel,
        mesh=mesh,
        out_type=jax.ShapeDtypeStruct((V, W), jnp.float32),
        scratch_types=[
            pltpu.VMEM((2, D, CH), jnp.float32),
            pltpu.VMEM((2, CH, W), jnp.float32),
            pltpu.SemaphoreType.DMA,
            pltpu.SemaphoreType.DMA,
        ],
        compiler_params=pltpu.CompilerParams(
            use_tc_tiling_on_sc=True, needs_layout_passes=False
        ),
    )
    def tpose_kernel(wt_hbm, wtail_hbm, out_hbm, bin_v, bout_v, sem0, sem1):
        sems = (sem0, sem1)
        wid = lax.axis_index("s") * 2 + lax.axis_index("c")
        nblk = jnp.where(wid < NTB - 244 * NW, 245, 244)
        lane = jnp.arange(16, dtype=jnp.int32)

        def start_in(k, p):
            t0 = (wid + k * NW) * CH
            pltpu.async_copy(
                wt_hbm.at[:, pl.ds(t0, CH)], bin_v.at[p], sems[p]
            )

        def wait_in(p):
            pltpu.make_async_copy(
                wt_hbm.at[:, pl.ds(0, CH)], bin_v.at[p], sems[p]
            ).wait()

        start_in(0, 0)
        start_in(1, 1)

        def group(g, carry):
            for u in (0, 1):
                k = jnp.minimum(2 * g + u, nblk - 1)
                wait_in(u)

                def row(r, c2):
                    ridx = jnp.zeros((16,), jnp.int32) + r
                    for m in range(4):
                        v = plsc.load_gather(
                            bin_v.at[u], [16 * m + lane, ridx]
                        )
                        bout_v[u, r, pl.ds(16 * m, 16)] = v
                    return c2

                lax.fori_loop(0, CH, row, 0)
                nk = jnp.minimum(2 * g + u + 2, nblk - 1)
                start_in(nk, u)
                t0 = (wid + k * NW) * CH
                pltpu.sync_copy(bout_v.at[u], out_hbm.at[pl.ds(t0, CH)])
            return carry

        lax.fori_loop(0, 123, group, 0)
        wait_in(0)
        wait_in(1)

        @pl.when(wid == 0)
        def _partial():
            pltpu.sync_copy(wtail_hbm, bin_v.at[0])

            def prow(r, c2):
                ridx = jnp.zeros((16,), jnp.int32) + r
                for m in range(4):
                    v = plsc.load_gather(bin_v.at[0], [16 * m + lane, ridx])
                    bout_v[0, r, pl.ds(16 * m, 16)] = v
                return c2

            lax.fori_loop(0, VREM, prow, 0)
            pltpu.sync_copy(
                bout_v.at[0, pl.ds(0, VREM)],
                out_hbm.at[pl.ds(NTB * CH, VREM)],
            )

    return tpose_kernel


def _build_gather():
    mesh = plsc.VectorSubcoreMesh(core_axis_name="c", subcore_axis_name="s")

    @functools.partial(
        pl.kernel,
        mesh=mesh,
        out_type=jax.ShapeDtypeStruct((B, W), jnp.float32),
        scratch_types=[
            pltpu.VMEM((NCH, CH), jnp.int32),
            pltpu.VMEM((2, CH, W), jnp.float32),
            pltpu.SemaphoreType.DMA,
            pltpu.SemaphoreType.DMA,
        ],
        compiler_params=pltpu.CompilerParams(use_tc_tiling_on_sc=False),
    )
    def emb_kernel(idx_hbm, table_hbm, out_hbm, idx_v, rows_v, sem0, sem1):
        sems = (sem0, sem1)
        wid = lax.axis_index("s") * 2 + lax.axis_index("c")
        base = wid * BPW
        pltpu.sync_copy(idx_hbm.at[wid], idx_v)

        pltpu.async_copy(table_hbm.at[idx_v.at[0]], rows_v.at[0], sem0)
        pltpu.async_copy(table_hbm.at[idx_v.at[1]], rows_v.at[1], sem1)

        def group(g, carry):
            for b in (0, 1):
                j = 2 * g + b
                pltpu.make_async_copy(
                    table_hbm.at[idx_v.at[0]], rows_v.at[b], sems[b]
                ).wait()
                pltpu.sync_copy(
                    rows_v.at[b], out_hbm.at[pl.ds(base + j * CH, CH)]
                )
                nxt = jnp.minimum(j + 2, NCH - 1)
                pltpu.async_copy(table_hbm.at[idx_v.at[nxt]], rows_v.at[b], sems[b])
            return carry

        lax.fori_loop(0, NCH // 2, group, 0)
        pltpu.make_async_copy(table_hbm.at[idx_v.at[0]], rows_v.at[0], sem0).wait()
        pltpu.make_async_copy(table_hbm.at[idx_v.at[0]], rows_v.at[1], sem1).wait()

    return emb_kernel


_tpose = _build_tpose()
_emb = _build_gather()


@jax.jit
def kernel(token_ids, weight):
    wt = weight.T
    wtail = jnp.pad(wt[:, NTB * CH:], ((0, 0), (0, W - VREM)))
    table = _tpose(wt, wtail)
    idxf = token_ids.T.reshape(NW, NCH, CH).astype(jnp.int32)
    out2 = _emb(idxf, table)
    out3 = out2.reshape(FIELDS, BATCH, W)[:, :, :D]
    return out3.transpose(1, 0, 2)


# speedup vs baseline: 2.4572x; 2.4572x over previous
import functools

import jax
import jax.numpy as jnp
from jax import lax
from jax.experimental import pallas as pl
from jax.experimental.pallas import tpu as pltpu
from jax.experimental.pallas import tpu_sc as plsc

BATCH = 16384
FIELDS = 26
D = 64
W = 128
B = BATCH * FIELDS
NW = 32
BPW = B // NW
CH = 128
NCH = BPW // CH


def _build():
    mesh = plsc.VectorSubcoreMesh(core_axis_name="c", subcore_axis_name="s")

    @functools.partial(
        pl.kernel,
        mesh=mesh,
        out_type=jax.ShapeDtypeStruct((B, W), jnp.float32),
        scratch_types=[
            pltpu.VMEM((NCH, CH), jnp.int32),
            pltpu.VMEM((2, CH, D), jnp.float32),
            pltpu.SemaphoreType.DMA,
            pltpu.SemaphoreType.DMA,
        ],
        compiler_params=pltpu.CompilerParams(use_tc_tiling_on_sc=False),
    )
    def emb_kernel(idx_hbm, table_hbm, out_hbm, idx_v, rows_v, sem0, sem1):
        sems = (sem0, sem1)
        wid = lax.axis_index("s") * 2 + lax.axis_index("c")
        base = wid * BPW
        pltpu.sync_copy(idx_hbm.at[wid], idx_v)

        pltpu.async_copy(table_hbm.at[idx_v.at[0]], rows_v.at[0], sem0)
        pltpu.async_copy(table_hbm.at[idx_v.at[1]], rows_v.at[1], sem1)

        def group(g, carry):
            for b in (0, 1):
                j = 2 * g + b
                pltpu.make_async_copy(
                    table_hbm.at[idx_v.at[0]], rows_v.at[b], sems[b]
                ).wait()
                pltpu.sync_copy(
                    rows_v.at[b],
                    out_hbm.at[pl.ds(base + j * CH, CH), pl.ds(0, D)],
                )
                nxt = jnp.minimum(j + 2, NCH - 1)
                pltpu.async_copy(table_hbm.at[idx_v.at[nxt]], rows_v.at[b], sems[b])
            return carry

        lax.fori_loop(0, NCH // 2, group, 0)
        pltpu.make_async_copy(table_hbm.at[idx_v.at[0]], rows_v.at[0], sem0).wait()
        pltpu.make_async_copy(table_hbm.at[idx_v.at[0]], rows_v.at[1], sem1).wait()

    return emb_kernel


_emb = _build()


@jax.jit
def kernel(token_ids, weight):
    idxf = token_ids.T.reshape(NW, NCH, CH).astype(jnp.int32)
    out2 = _emb(idxf, weight)
    out3 = out2.reshape(FIELDS, BATCH, W)[:, :, :D]
    return out3.transpose(1, 0, 2)


# speedup vs baseline: 2.4971x; 1.0163x over previous
import functools

import jax
import jax.numpy as jnp
from jax import lax
from jax.experimental import pallas as pl
from jax.experimental.pallas import tpu as pltpu
from jax.experimental.pallas import tpu_sc as plsc

BATCH = 16384
FIELDS = 26
D = 64
W = 128
B = BATCH * FIELDS
NW = 32
BPW = B // NW
CH = 512
NCH = BPW // CH


def _build():
    mesh = plsc.VectorSubcoreMesh(core_axis_name="c", subcore_axis_name="s")

    @functools.partial(
        pl.kernel,
        mesh=mesh,
        out_type=jax.ShapeDtypeStruct((B, W), jnp.float32),
        scratch_types=[
            pltpu.VMEM((NCH, CH), jnp.int32),
            pltpu.VMEM((2, CH, D), jnp.float32),
            pltpu.SemaphoreType.DMA,
            pltpu.SemaphoreType.DMA,
        ],
        compiler_params=pltpu.CompilerParams(use_tc_tiling_on_sc=False),
    )
    def emb_kernel(idx_hbm, table_hbm, out_hbm, idx_v, rows_v, sem0, sem1):
        sems = (sem0, sem1)
        wid = lax.axis_index("s") * 2 + lax.axis_index("c")
        base = wid * BPW
        pltpu.sync_copy(idx_hbm.at[wid], idx_v)

        pltpu.async_copy(table_hbm.at[idx_v.at[0]], rows_v.at[0], sem0)
        pltpu.async_copy(table_hbm.at[idx_v.at[1]], rows_v.at[1], sem1)

        def group(g, carry):
            for b in (0, 1):
                j = 2 * g + b
                pltpu.make_async_copy(
                    table_hbm.at[idx_v.at[0]], rows_v.at[b], sems[b]
                ).wait()
                pltpu.sync_copy(
                    rows_v.at[b],
                    out_hbm.at[pl.ds(base + j * CH, CH), pl.ds(0, D)],
                )
                nxt = jnp.minimum(j + 2, NCH - 1)
                pltpu.async_copy(table_hbm.at[idx_v.at[nxt]], rows_v.at[b], sems[b])
            return carry

        lax.fori_loop(0, NCH // 2, group, 0)
        pltpu.make_async_copy(table_hbm.at[idx_v.at[0]], rows_v.at[0], sem0).wait()
        pltpu.make_async_copy(table_hbm.at[idx_v.at[0]], rows_v.at[1], sem1).wait()

    return emb_kernel


_emb = _build()


@jax.jit
def kernel(token_ids, weight):
    idxf = token_ids.T.reshape(NW, NCH, CH).astype(jnp.int32)
    out2 = _emb(idxf, weight)
    out3 = out2.reshape(FIELDS, BATCH, W)[:, :, :D]
    return out3.transpose(1, 0, 2)
